# baseline (speedup 1.0000x reference)
import jax
import jax.numpy as jnp
from jax import lax
from jax.experimental import pallas as pl
from jax.experimental.pallas import tpu as pltpu
from jax.experimental.pallas import tpu_sc as plsc

MAX2D = 501
D = 1024
DH = D // 2
TOKENS = 32768
NW = 32
TPW = TOKENS // NW
CHUNK = 16
NCHUNK = TPW // CHUNK
LANES = 16
NTRIP = (NCHUNK - 2) // 3
NREM = NCHUNK - 3 * NTRIP


def _sc_body(bbox_hbm, table_hbm, out_hbm, bbox_v, idx_v, r0, r1, r2,
             g0, g1, g2, o0, o1, o2):
    wid = lax.axis_index("c") * 16 + lax.axis_index("s")
    bufs = (r0, r1, r2)
    gsems = (g0, g1, g2)
    osems = (o0, o1, o2)

    pltpu.sync_copy(bbox_hbm.at[wid], bbox_v)

    yoff = (lax.iota(jnp.int32, LANES) & 1) * MAX2D

    def compute_idx(i, _):
        v = bbox_v[pl.ds(i * LANES, LANES)]
        v = jnp.minimum(jnp.maximum(v, 0.0), 1.0)
        idx_v[pl.ds(i * LANES, LANES)] = (
            (v * float(MAX2D - 1)).astype(jnp.int32) + yoff
        )
        return 0

    lax.fori_loop(0, 4 * TPW // LANES, compute_idx, 0)

    def gather(c, par):
        for u in range(4):
            pltpu.async_copy(
                table_hbm.at[idx_v.at[pl.ds(c * 4 * CHUNK + u * CHUNK,
                                            CHUNK)]],
                bufs[par].at[pl.ds(u * CHUNK, CHUNK)], gsems[par],
            )

    def unpack2(w):
        lo = lax.bitcast_convert_type(w << 16, jnp.float32)
        hi = lax.bitcast_convert_type(w, jnp.float32)
        return lo, hi

    def fold_and_out(c, par):
        buf = bufs[par]
        pltpu.make_async_copy(
            table_hbm.at[pl.ds(0, 4 * CHUNK)], buf, gsems[par]
        ).wait()

        def fold(t, _):
            for k in range(DH // LANES):
                s = pl.ds(k * LANES, LANES)
                a, b = unpack2(buf[4 * t, s])
                e, f = unpack2(buf[4 * t + 1, s])
                g, h = unpack2(buf[4 * t + 2, s])
                p, q = unpack2(buf[4 * t + 3, s])
                buf[2 * t, s] = lax.bitcast_convert_type(
                    (a + e) + (g + p), jnp.int32)
                buf[2 * t + 1, s] = lax.bitcast_convert_type(
                    (b + f) + (h + q), jnp.int32)
            return 0

        lax.fori_loop(0, CHUNK, fold, 0)
        pltpu.async_copy(
            buf.at[pl.ds(0, 2 * CHUNK)],
            out_hbm.at[pl.ds(2 * (wid * TPW + c * CHUNK), 2 * CHUNK)],
            osems[par],
        )

    def drain_out(par):
        pltpu.make_async_copy(
            bufs[par].at[pl.ds(0, 2 * CHUNK)],
            out_hbm.at[pl.ds(0, 2 * CHUNK)], osems[par],
        ).wait()

    gather(0, 0)

    def do_triple(c3, _):
        for par in range(3):
            c = c3 * 3 + par
            parn = (par + 1) % 3
            if par == 2:
                drain_out(parn)
            else:
                @pl.when(c3 > 0)
                def _():
                    drain_out(parn)
            gather(c + 1, parn)
            fold_and_out(c, par)
        return 0

    lax.fori_loop(0, NTRIP, do_triple, 0)

    for e in range(NREM):
        c = NTRIP * 3 + e
        if c + 1 < NCHUNK:
            parn = (c + 1) % 3
            drain_out(parn)
            gather(c + 1, parn)
        fold_and_out(c, c % 3)
    for c in range(NCHUNK - 3, NCHUNK):
        drain_out(c % 3)


@jax.jit
def _cell_embed(bbox_blocks, table_packed):
    mesh = plsc.VectorSubcoreMesh(
        core_axis_name="c", subcore_axis_name="s", num_cores=2, num_subcores=16
    )
    return pl.kernel(
        _sc_body,
        out_type=jax.ShapeDtypeStruct((2 * TOKENS, DH), jnp.int32),
        mesh=mesh,
        scratch_types=[
            pltpu.VMEM((4 * TPW,), jnp.float32),
            pltpu.VMEM((4 * TPW,), jnp.int32),
            pltpu.VMEM((4 * CHUNK, DH), jnp.int32),
            pltpu.VMEM((4 * CHUNK, DH), jnp.int32),
            pltpu.VMEM((4 * CHUNK, DH), jnp.int32),
            pltpu.SemaphoreType.DMA,
            pltpu.SemaphoreType.DMA,
            pltpu.SemaphoreType.DMA,
            pltpu.SemaphoreType.DMA,
            pltpu.SemaphoreType.DMA,
            pltpu.SemaphoreType.DMA,
        ],
    )(bbox_blocks, table_packed)


def kernel(bbox, x_emb, y_emb):
    b, s, _ = bbox.shape
    table = jnp.concatenate([x_emb, y_emb], axis=0).astype(jnp.bfloat16)
    halves = jnp.stack([table[:, :DH], table[:, DH:]], axis=-1)
    table_packed = lax.bitcast_convert_type(
        lax.bitcast_convert_type(halves, jnp.int16), jnp.int32
    )
    bbox_blocks = bbox.reshape(NW, 4 * TPW)
    out = _cell_embed(bbox_blocks, table_packed)
    return lax.bitcast_convert_type(out, jnp.float32).reshape(b, s, D)


# speedup vs baseline: 1.7717x; 1.7717x over previous
import jax
import jax.numpy as jnp
from jax import lax
from jax.experimental import pallas as pl
from jax.experimental.pallas import tpu as pltpu
from jax.experimental.pallas import tpu_sc as plsc

MAX2D = 501
D = 1024
TOKENS = 32768
NW = 32
TPW = TOKENS // NW
CHUNK = 8
NCHUNK = TPW // CHUNK
LANES = 16
NTRIP = (NCHUNK - 2) // 3


def _sc_body(bbox_hbm, table_hbm, out_hbm, idx_v, r0, r1, r2,
             g0, g1, g2, o0, o1, o2):
    wid = lax.axis_index("s") * 2 + lax.axis_index("c")
    bufs = (r0, r1, r2)
    gsems = (g0, g1, g2)
    osems = (o0, o1, o2)

    pltpu.sync_copy(bbox_hbm.at[wid], r0.at[pl.ds(0, 4)])

    for j in range(4):
        off = 0 if j % 2 == 0 else MAX2D

        def compute_idx(i, _, j=j, off=off):
            v = r0[j, pl.ds(i * LANES, LANES)]
            v = jnp.minimum(jnp.maximum(v, 0.0), 1.0)
            idx = (v * float(MAX2D - 1)).astype(jnp.int32) + off
            idx_v[pl.ds(j * TPW + i * LANES, LANES)] = idx
            return 0

        lax.fori_loop(0, TPW // LANES, compute_idx, 0)

    def gather(c, par):
        for j in range(4):
            pltpu.async_copy(
                table_hbm.at[idx_v.at[pl.ds(j * TPW + c * CHUNK, CHUNK)]],
                bufs[par].at[pl.ds(j * CHUNK, CHUNK)], gsems[par],
            )

    def fold_and_out(c, par):
        buf = bufs[par]
        pltpu.make_async_copy(
            table_hbm.at[pl.ds(0, 4 * CHUNK)], buf, gsems[par]
        ).wait()

        def fold(t, _):
            for k in range(D // LANES):
                s = pl.ds(k * LANES, LANES)
                v = buf[CHUNK + t, s] + buf[2 * CHUNK + t, s]
                v = v + buf[3 * CHUNK + t, s]
                plsc.addupdate(buf.at[t, s], v)
            return 0

        lax.fori_loop(0, CHUNK, fold, 0)
        pltpu.async_copy(
            buf.at[pl.ds(0, CHUNK)],
            out_hbm.at[pl.ds(wid * TPW + c * CHUNK, CHUNK)],
            osems[par],
        )

    def drain_out(par):
        pltpu.make_async_copy(
            bufs[par].at[pl.ds(0, CHUNK)], out_hbm.at[pl.ds(0, CHUNK)],
            osems[par],
        ).wait()

    gather(0, 0)

    def do_triple(c3, _):
        for par in range(3):
            c = c3 * 3 + par
            parn = (par + 1) % 3
            if par == 2:
                drain_out(parn)
            else:
                @pl.when(c3 > 0)
                def _():
                    drain_out(parn)
            gather(c + 1, parn)
            fold_and_out(c, par)
        return 0

    lax.fori_loop(0, NTRIP, do_triple, 0)

    c = NTRIP * 3
    drain_out(1)
    gather(c + 1, 1)
    fold_and_out(c, 0)
    fold_and_out(c + 1, 1)
    drain_out(2)
    drain_out(0)
    drain_out(1)


@jax.jit
def _cell_embed(bbox_blocks, table):
    mesh = plsc.VectorSubcoreMesh(
        core_axis_name="c", subcore_axis_name="s", num_cores=2, num_subcores=16
    )
    return pl.kernel(
        _sc_body,
        out_type=jax.ShapeDtypeStruct((TOKENS, D), jnp.float32),
        mesh=mesh,
        scratch_types=[
            pltpu.VMEM((TPW * 4,), jnp.int32),
            pltpu.VMEM((4 * CHUNK, D), jnp.float32),
            pltpu.VMEM((4 * CHUNK, D), jnp.float32),
            pltpu.VMEM((4 * CHUNK, D), jnp.float32),
            pltpu.SemaphoreType.DMA,
            pltpu.SemaphoreType.DMA,
            pltpu.SemaphoreType.DMA,
            pltpu.SemaphoreType.DMA,
            pltpu.SemaphoreType.DMA,
            pltpu.SemaphoreType.DMA,
        ],
    )(bbox_blocks, table)


def kernel(bbox, x_emb, y_emb):
    b, s, _ = bbox.shape
    table = jnp.concatenate([x_emb, y_emb], axis=0)
    bbox_blocks = (
        bbox.reshape(-1, 4).T.reshape(4, NW, TPW).transpose(1, 0, 2)
    )
    out = _cell_embed(bbox_blocks, table)
    return out.reshape(b, s, D)
